# Initial kernel scaffold; baseline (speedup 1.0000x reference)
import jax
import jax.numpy as jnp
from jax.experimental import pallas as pl


def kernel(x, edge_index, edge_attr, batch, params):
    raise NotImplementedError("write your pallas kernel here")


# speedup vs baseline: 2.6057x; 2.6057x over previous
import functools

import jax
import jax.numpy as jnp
from jax import lax
from jax.experimental import pallas as pl
from jax.experimental.pallas import tpu as pltpu
from jax.experimental.pallas import tpu_sc as plsc

_NC, _NS = 2, 16
_NW = _NC * _NS
_NG = 64


def _edge_mlp(attr, w1, b1, w2, b2):
    e = attr.shape[0]
    dout = w2.shape[1]
    be = 4000

    def body(a_ref, w1_ref, b1_ref, w2_ref, b2_ref, o_ref):
        a = a_ref[...]
        h = jnp.maximum(
            jnp.dot(a, w1_ref[...], preferred_element_type=jnp.float32)
            + b1_ref[...], 0.0)
        o_ref[...] = (
            jnp.dot(h, w2_ref[...], preferred_element_type=jnp.float32)
            + b2_ref[...])

    return pl.pallas_call(
        body,
        grid=(e // be,),
        in_specs=[
            pl.BlockSpec((be, attr.shape[1]), lambda i: (i, 0)),
            pl.BlockSpec(w1.shape, lambda i: (0, 0)),
            pl.BlockSpec(b1.shape, lambda i: (0, 0)),
            pl.BlockSpec(w2.shape, lambda i: (0, 0)),
            pl.BlockSpec(b2.shape, lambda i: (0, 0)),
        ],
        out_specs=pl.BlockSpec((be, dout), lambda i: (i, 0)),
        out_shape=jax.ShapeDtypeStruct((e, dout), jnp.float32),
    )(attr, w1, b1, w2, b2)


def _sc_msgpass(x, ee, src, dst):
    n, d = x.shape
    e = src.shape[0]
    ew = e // _NW
    c_sz = 80
    nch = ew // c_sz
    rpt, zc = 624, 208

    mesh = plsc.VectorSubcoreMesh(
        core_axis_name="c", subcore_axis_name="s",
        num_cores=_NC, num_subcores=_NS)

    @functools.partial(
        pl.kernel,
        out_type=jax.ShapeDtypeStruct((_NC, n, d), jnp.float32),
        mesh=mesh,
        scratch_types=[
            pltpu.VMEM((c_sz,), jnp.int32),
            pltpu.VMEM((c_sz,), jnp.int32),
            pltpu.VMEM((c_sz, d), jnp.float32),
            pltpu.VMEM((c_sz, d), jnp.float32),
            pltpu.VMEM((zc, d), jnp.float32),
            pltpu.VMEM_SHARED((n, d), jnp.float32),
            pltpu.SemaphoreType.DMA,
        ],
    )
    def k(x_hbm, ee_hbm, src_hbm, dst_hbm, out_hbm,
          src_v, dst_v, ee_v, rows_v, zero_v, agg_sh, sem):
        ci = lax.axis_index("c")
        si = lax.axis_index("s")
        wid = si * _NC + ci

        def zrow(i, _):
            for j in range(d // 16):
                zero_v[i, pl.ds(j * 16, 16)] = jnp.zeros((16,), jnp.float32)
            return 0

        lax.fori_loop(0, zc, zrow, 0)
        for z in range(rpt // zc):
            pltpu.sync_copy(zero_v,
                            agg_sh.at[pl.ds(si * rpt + z * zc, zc)])

        @pl.when(si == _NS - 1)
        def _():
            pltpu.sync_copy(zero_v.at[pl.ds(0, n - _NS * rpt)],
                            agg_sh.at[pl.ds(_NS * rpt, n - _NS * rpt)])

        plsc.subcore_barrier()

        base = wid * ew

        def chunk(t, _):
            off = base + t * c_sz
            pltpu.sync_copy(src_hbm.at[pl.ds(off, c_sz)], src_v)
            pltpu.sync_copy(dst_hbm.at[pl.ds(off, c_sz)], dst_v)
            pltpu.sync_copy(ee_hbm.at[pl.ds(off, c_sz)], ee_v)
            pltpu.async_copy(x_hbm.at[src_v], rows_v, sem).wait()

            def msg(i, _):
                for j in range(d // 16):
                    sl = pl.ds(j * 16, 16)
                    rows_v[i, sl] = jnp.maximum(rows_v[i, sl] + ee_v[i, sl],
                                                0.0)
                return 0

            lax.fori_loop(0, c_sz, msg, 0)
            pltpu.sync_copy(rows_v, agg_sh.at[dst_v], add=True)
            return 0

        lax.fori_loop(0, nch, chunk, 0)
        plsc.subcore_barrier()

        for z in range(rpt // zc):
            row0 = si * rpt + z * zc
            pltpu.sync_copy(agg_sh.at[pl.ds(row0, zc)],
                            out_hbm.at[ci, pl.ds(row0, zc)])

        @pl.when(si == _NS - 1)
        def _():
            pltpu.sync_copy(agg_sh.at[pl.ds(_NS * rpt, n - _NS * rpt)],
                            out_hbm.at[ci, pl.ds(_NS * rpt, n - _NS * rpt)])

    return k(x, ee, src, dst)


def _node_stats(xin, agg0, agg1, w1, b1, w2, b2, eps11):
    n, din = xin.shape
    d2 = w2.shape[1]
    nb = 1000
    nblk = n // nb

    def body(x_ref, a0_ref, a1_ref, w1_ref, b1_ref, w2_ref, b2_ref, eps_ref,
             y_ref, st_ref):
        i = pl.program_id(0)
        h = (x_ref[...] * (1.0 + eps_ref[0, 0])
             + a0_ref[...] + a1_ref[...])
        t = jnp.maximum(
            jnp.dot(h, w1_ref[...], preferred_element_type=jnp.float32)
            + b1_ref[...], 0.0)
        y = jnp.maximum(
            jnp.dot(t, w2_ref[...], preferred_element_type=jnp.float32)
            + b2_ref[...], 0.0)
        y_ref[...] = y

        @pl.when(i == 0)
        def _():
            st_ref[...] = jnp.zeros_like(st_ref)

        st_ref[0:1, :] += jnp.sum(y, axis=0, keepdims=True)
        st_ref[1:2, :] += jnp.sum(y * y, axis=0, keepdims=True)

    return pl.pallas_call(
        body,
        grid=(nblk,),
        in_specs=[
            pl.BlockSpec((nb, din), lambda i: (i, 0)),
            pl.BlockSpec((nb, din), lambda i: (i, 0)),
            pl.BlockSpec((nb, din), lambda i: (i, 0)),
            pl.BlockSpec(w1.shape, lambda i: (0, 0)),
            pl.BlockSpec(b1.shape, lambda i: (0, 0)),
            pl.BlockSpec(w2.shape, lambda i: (0, 0)),
            pl.BlockSpec(b2.shape, lambda i: (0, 0)),
            pl.BlockSpec((1, 1), lambda i: (0, 0)),
        ],
        out_specs=[
            pl.BlockSpec((nb, d2), lambda i: (i, 0)),
            pl.BlockSpec((8, d2), lambda i: (0, 0)),
        ],
        out_shape=[
            jax.ShapeDtypeStruct((n, d2), jnp.float32),
            jax.ShapeDtypeStruct((8, d2), jnp.float32),
        ],
    )(xin, agg0, agg1, w1, b1, w2, b2, eps11)


def _bn_apply(y, st, g, b):
    n, d2 = y.shape
    nb = 2000

    def body(y_ref, st_ref, g_ref, b_ref, o_ref):
        mean = st_ref[0:1, :] / n
        var = st_ref[1:2, :] / n - mean * mean
        o_ref[...] = ((y_ref[...] - mean) * lax.rsqrt(var + 1e-5)
                      * g_ref[...] + b_ref[...])

    return pl.pallas_call(
        body,
        grid=(n // nb,),
        in_specs=[
            pl.BlockSpec((nb, d2), lambda i: (i, 0)),
            pl.BlockSpec((8, d2), lambda i: (0, 0)),
            pl.BlockSpec((1, d2), lambda i: (0, 0)),
            pl.BlockSpec((1, d2), lambda i: (0, 0)),
        ],
        out_specs=pl.BlockSpec((nb, d2), lambda i: (i, 0)),
        out_shape=jax.ShapeDtypeStruct((n, d2), jnp.float32),
    )(y, st, g, b)


def _pool_head(x1, x2, x3, x4, batch_row,
               w1, b1, w2, b2, w3, b3, w4, b4):
    n, d = x1.shape
    nb = 2000
    nblk = n // nb

    def body(b_ref, x1_ref, x2_ref, x3_ref, x4_ref,
             w1_ref, b1_ref, w2_ref, b2_ref, w3_ref, b3_ref, w4_ref, b4_ref,
             o_ref, acc_ref, cnt_ref):
        i = pl.program_id(0)

        @pl.when(i == 0)
        def _():
            acc_ref[...] = jnp.zeros_like(acc_ref)
            cnt_ref[...] = jnp.zeros_like(cnt_ref)

        seg = lax.broadcasted_iota(jnp.int32, (_NG, nb), 0)
        oh = (seg == b_ref[...].reshape(1, nb)).astype(jnp.float32)
        xcat = jnp.concatenate(
            [x1_ref[...], x2_ref[...], x3_ref[...], x4_ref[...]], axis=1)
        acc_ref[...] += lax.dot_general(
            oh, xcat, (((1,), (0,)), ((), ())),
            preferred_element_type=jnp.float32)
        cnt_ref[...] += jnp.sum(oh, axis=1, keepdims=True)

        @pl.when(i == nblk - 1)
        def _():
            pooled = acc_ref[...] / jnp.maximum(cnt_ref[...], 1.0)
            h1 = jnp.maximum(
                jnp.dot(pooled, w1_ref[...],
                        preferred_element_type=jnp.float32) + b1_ref[...], 0.0)
            h2 = jnp.maximum(
                jnp.dot(h1, w2_ref[...],
                        preferred_element_type=jnp.float32) + b2_ref[...], 0.0)
            h3 = jnp.maximum(
                jnp.dot(h2, w3_ref[...],
                        preferred_element_type=jnp.float32) + b3_ref[...], 0.0)
            z = (jnp.dot(h3, w4_ref[...],
                         preferred_element_type=jnp.float32) + b4_ref[...])
            m = jnp.max(z, axis=1, keepdims=True)
            lse = m + jnp.log(jnp.sum(jnp.exp(z - m), axis=1, keepdims=True))
            o_ref[...] = z - lse

    return pl.pallas_call(
        body,
        grid=(nblk,),
        in_specs=[
            pl.BlockSpec((1, 1, nb), lambda i: (i, 0, 0)),
            pl.BlockSpec((nb, d), lambda i: (i, 0)),
            pl.BlockSpec((nb, d), lambda i: (i, 0)),
            pl.BlockSpec((nb, d), lambda i: (i, 0)),
            pl.BlockSpec((nb, d), lambda i: (i, 0)),
            pl.BlockSpec(w1.shape, lambda i: (0, 0)),
            pl.BlockSpec(b1.shape, lambda i: (0, 0)),
            pl.BlockSpec(w2.shape, lambda i: (0, 0)),
            pl.BlockSpec(b2.shape, lambda i: (0, 0)),
            pl.BlockSpec(w3.shape, lambda i: (0, 0)),
            pl.BlockSpec(b3.shape, lambda i: (0, 0)),
            pl.BlockSpec(w4.shape, lambda i: (0, 0)),
            pl.BlockSpec(b4.shape, lambda i: (0, 0)),
        ],
        out_specs=pl.BlockSpec((_NG, 2), lambda i: (0, 0)),
        out_shape=jax.ShapeDtypeStruct((_NG, 2), jnp.float32),
        scratch_shapes=[
            pltpu.VMEM((_NG, 4 * d), jnp.float32),
            pltpu.VMEM((_NG, 1), jnp.float32),
        ],
    )(batch_row, x1, x2, x3, x4, w1, b1, w2, b2, w3, b3, w4, b4)


def _pad2(w, r, c):
    return jnp.pad(w, ((0, r - w.shape[0]), (0, c - w.shape[1])))


def _padb(b, c):
    return jnp.pad(b, (0, c - b.shape[0])).reshape(1, -1)


def kernel(x, edge_index, edge_attr, batch, params):
    p = params
    src = edge_index[0]
    dst = edge_index[1]

    c1 = p["conv1"]
    ee1 = _edge_mlp(edge_attr,
                    _pad2(c1["be1"]["W"], 3, 16), _padb(c1["be1"]["b"], 16),
                    _pad2(c1["be2"]["W"], 16, 128), _padb(c1["be2"]["b"], 128))
    ees = [
        _edge_mlp(edge_attr, cv["be1"]["W"], cv["be1"]["b"].reshape(1, -1),
                  cv["be2"]["W"], cv["be2"]["b"].reshape(1, -1))
        for cv in (p["conv2"], p["conv3"], p["conv4"])
    ]

    x128 = jnp.pad(x, ((0, 0), (0, 128 - x.shape[1])))

    def layer(xin, cv, ee, bn, pad_in):
        ag = _sc_msgpass(xin, ee, src, dst)
        if pad_in:
            w1 = _pad2(cv["m1"]["W"], 128, 16)
            b1 = _padb(cv["m1"]["b"], 16)
            w2 = _pad2(cv["m2"]["W"], 16, 128)
        else:
            w1 = cv["m1"]["W"]
            b1 = cv["m1"]["b"].reshape(1, -1)
            w2 = cv["m2"]["W"]
        b2 = cv["m2"]["b"].reshape(1, -1)
        y, st = _node_stats(xin, ag[0], ag[1], w1, b1, w2, b2,
                            cv["eps"].reshape(1, 1))
        return _bn_apply(y, st, bn["g"].reshape(1, -1), bn["b"].reshape(1, -1))

    x1r = layer(x128, p["conv1"], ee1, p["bn1"], True)
    x2r = layer(x1r, p["conv2"], ees[0], p["bn2"], False)
    x3r = layer(x2r, p["conv3"], ees[1], p["bn3"], False)
    x4r = layer(x3r, p["conv4"], ees[2], p["bn4"], False)

    return _pool_head(
        x1r, x2r, x3r, x4r, batch.reshape(-1, 1, 2000),
        p["fc1"]["W"], p["fc1"]["b"].reshape(1, -1),
        p["fc2"]["W"], p["fc2"]["b"].reshape(1, -1),
        p["fc3"]["W"], p["fc3"]["b"].reshape(1, -1),
        p["fc4"]["W"], p["fc4"]["b"].reshape(1, -1))


# speedup vs baseline: 4.7361x; 1.8176x over previous
import functools

import jax
import jax.numpy as jnp
from jax import lax
from jax.experimental import pallas as pl
from jax.experimental.pallas import tpu as pltpu
from jax.experimental.pallas import tpu_sc as plsc

_NC, _NS = 2, 16
_NW = _NC * _NS
_NG = 64


def _edge_mlp(attr, w1, b1, w2, b2):
    e = attr.shape[0]
    dout = w2.shape[1]
    be = 4000

    def body(a_ref, w1_ref, b1_ref, w2_ref, b2_ref, o_ref):
        a = a_ref[...]
        h = jnp.maximum(
            jnp.dot(a, w1_ref[...], preferred_element_type=jnp.float32)
            + b1_ref[...], 0.0)
        o_ref[...] = (
            jnp.dot(h, w2_ref[...], preferred_element_type=jnp.float32)
            + b2_ref[...])

    return pl.pallas_call(
        body,
        grid=(e // be,),
        in_specs=[
            pl.BlockSpec((be, attr.shape[1]), lambda i: (i, 0)),
            pl.BlockSpec(w1.shape, lambda i: (0, 0)),
            pl.BlockSpec(b1.shape, lambda i: (0, 0)),
            pl.BlockSpec(w2.shape, lambda i: (0, 0)),
            pl.BlockSpec(b2.shape, lambda i: (0, 0)),
        ],
        out_specs=pl.BlockSpec((be, dout), lambda i: (i, 0)),
        out_shape=jax.ShapeDtypeStruct((e, dout), jnp.float32),
    )(attr, w1, b1, w2, b2)


def _sc_msgpass(x, ee, eic):
    n, d = x.shape
    e = eic.shape[0] * eic.shape[2]
    ew = e // _NW
    c_sz = eic.shape[2]
    nch = ew // c_sz
    rpt, zc = 624, 16

    mesh = plsc.VectorSubcoreMesh(
        core_axis_name="c", subcore_axis_name="s",
        num_cores=_NC, num_subcores=_NS)

    @functools.partial(
        pl.kernel,
        out_type=jax.ShapeDtypeStruct((_NC, n, d), jnp.float32),
        mesh=mesh,
        scratch_types=[
            pltpu.VMEM((2, 2, c_sz), jnp.int32),
            pltpu.VMEM((2, c_sz, d), jnp.float32),
            pltpu.VMEM((2, c_sz, d), jnp.float32),
            pltpu.VMEM((zc, d), jnp.float32),
            pltpu.VMEM_SHARED((n, d), jnp.float32),
            pltpu.SemaphoreType.DMA,
            pltpu.SemaphoreType.DMA,
            pltpu.SemaphoreType.DMA,
            pltpu.SemaphoreType.DMA,
            pltpu.SemaphoreType.DMA,
        ],
    )
    def k(x_hbm, ee_hbm, ei_hbm, out_hbm,
          idx_v, ee_v, rows_v, zero_v, agg_sh, es0, es1, gs0, gs1, dsem):
        ci = lax.axis_index("c")
        si = lax.axis_index("s")
        wid = si * _NC + ci

        def zrow(i, _):
            for j in range(d // 16):
                zero_v[i, pl.ds(j * 16, 16)] = jnp.zeros((16,), jnp.float32)
            return 0

        lax.fori_loop(0, zc, zrow, 0)
        for z in range(rpt // zc):
            pltpu.async_copy(zero_v,
                             agg_sh.at[pl.ds(si * rpt + z * zc, zc)], dsem)

        @pl.when(si == _NS - 1)
        def _():
            pltpu.async_copy(zero_v,
                             agg_sh.at[pl.ds(_NS * rpt, n - _NS * rpt)], dsem)

        for z in range(rpt // zc):
            pltpu.make_async_copy(
                zero_v, agg_sh.at[pl.ds(z * zc, zc)], dsem).wait()

        @pl.when(si == _NS - 1)
        def _():
            pltpu.make_async_copy(
                zero_v, agg_sh.at[pl.ds(0, zc)], dsem).wait()

        plsc.subcore_barrier()

        base = wid * ew
        base_g = wid * nch
        sems = ((es0, gs0), (es1, gs1))

        def start_fetch(t, b):
            es, gs = sems[b]
            pltpu.async_copy(ee_hbm.at[pl.ds(base + t * c_sz, c_sz)],
                             ee_v.at[b], es)
            pltpu.async_copy(x_hbm.at[idx_v.at[b, 0]], rows_v.at[b], gs)

        def finish_chunk(b):
            es, gs = sems[b]
            pltpu.make_async_copy(ee_hbm.at[pl.ds(0, c_sz)],
                                  ee_v.at[b], es).wait()
            pltpu.make_async_copy(ee_hbm.at[pl.ds(0, c_sz)],
                                  rows_v.at[b], gs).wait()

            def msg(i, _):
                for j in range(d // 16):
                    sl = pl.ds(j * 16, 16)
                    rows_v[b, i, sl] = jnp.maximum(
                        rows_v[b, i, sl] + ee_v[b, i, sl], 0.0)
                return 0

            lax.fori_loop(0, c_sz, msg, 0)
            pltpu.sync_copy(rows_v.at[b], agg_sh.at[idx_v.at[b, 1]], add=True)

        pltpu.sync_copy(ei_hbm.at[base_g], idx_v.at[0])
        start_fetch(0, 0)

        def pair(i, _):
            t0 = i * 2
            for db in (0, 1):
                tn = t0 + db + 1
                nb = 1 - db
                pltpu.sync_copy(ei_hbm.at[base_g + tn], idx_v.at[nb])
                start_fetch(tn, nb)
                finish_chunk(db)
            return 0

        lax.fori_loop(0, (nch - 1) // 2, pair, 0)
        finish_chunk((nch - 1) % 2)

        plsc.subcore_barrier()

        for z in range(3):
            row0 = si * rpt + z * 208
            pltpu.sync_copy(agg_sh.at[pl.ds(row0, 208)],
                            out_hbm.at[ci, pl.ds(row0, 208)])

        @pl.when(si == _NS - 1)
        def _():
            pltpu.sync_copy(agg_sh.at[pl.ds(_NS * rpt, n - _NS * rpt)],
                            out_hbm.at[ci, pl.ds(_NS * rpt, n - _NS * rpt)])

    return k(x, ee, eic)


def _node_stats(xin, agg0, agg1, w1, b1, w2, b2, eps11):
    n, din = xin.shape
    d2 = w2.shape[1]
    nb = 1000
    nblk = n // nb

    def body(x_ref, a0_ref, a1_ref, w1_ref, b1_ref, w2_ref, b2_ref, eps_ref,
             y_ref, st_ref):
        i = pl.program_id(0)
        h = (x_ref[...] * (1.0 + eps_ref[0, 0])
             + a0_ref[...] + a1_ref[...])
        t = jnp.maximum(
            jnp.dot(h, w1_ref[...], preferred_element_type=jnp.float32)
            + b1_ref[...], 0.0)
        y = jnp.maximum(
            jnp.dot(t, w2_ref[...], preferred_element_type=jnp.float32)
            + b2_ref[...], 0.0)
        y_ref[...] = y

        @pl.when(i == 0)
        def _():
            st_ref[...] = jnp.zeros_like(st_ref)

        st_ref[0:1, :] += jnp.sum(y, axis=0, keepdims=True)
        st_ref[1:2, :] += jnp.sum(y * y, axis=0, keepdims=True)

    return pl.pallas_call(
        body,
        grid=(nblk,),
        in_specs=[
            pl.BlockSpec((nb, din), lambda i: (i, 0)),
            pl.BlockSpec((nb, din), lambda i: (i, 0)),
            pl.BlockSpec((nb, din), lambda i: (i, 0)),
            pl.BlockSpec(w1.shape, lambda i: (0, 0)),
            pl.BlockSpec(b1.shape, lambda i: (0, 0)),
            pl.BlockSpec(w2.shape, lambda i: (0, 0)),
            pl.BlockSpec(b2.shape, lambda i: (0, 0)),
            pl.BlockSpec((1, 1), lambda i: (0, 0)),
        ],
        out_specs=[
            pl.BlockSpec((nb, d2), lambda i: (i, 0)),
            pl.BlockSpec((8, d2), lambda i: (0, 0)),
        ],
        out_shape=[
            jax.ShapeDtypeStruct((n, d2), jnp.float32),
            jax.ShapeDtypeStruct((8, d2), jnp.float32),
        ],
    )(xin, agg0, agg1, w1, b1, w2, b2, eps11)


def _bn_apply(y, st, g, b):
    n, d2 = y.shape
    nb = 2000

    def body(y_ref, st_ref, g_ref, b_ref, o_ref):
        mean = st_ref[0:1, :] / n
        var = st_ref[1:2, :] / n - mean * mean
        o_ref[...] = ((y_ref[...] - mean) * lax.rsqrt(var + 1e-5)
                      * g_ref[...] + b_ref[...])

    return pl.pallas_call(
        body,
        grid=(n // nb,),
        in_specs=[
            pl.BlockSpec((nb, d2), lambda i: (i, 0)),
            pl.BlockSpec((8, d2), lambda i: (0, 0)),
            pl.BlockSpec((1, d2), lambda i: (0, 0)),
            pl.BlockSpec((1, d2), lambda i: (0, 0)),
        ],
        out_specs=pl.BlockSpec((nb, d2), lambda i: (i, 0)),
        out_shape=jax.ShapeDtypeStruct((n, d2), jnp.float32),
    )(y, st, g, b)


def _pool_head(x1, x2, x3, x4, batch_row,
               w1, b1, w2, b2, w3, b3, w4, b4):
    n, d = x1.shape
    nb = 2000
    nblk = n // nb

    def body(b_ref, x1_ref, x2_ref, x3_ref, x4_ref,
             w1_ref, b1_ref, w2_ref, b2_ref, w3_ref, b3_ref, w4_ref, b4_ref,
             o_ref, acc_ref, cnt_ref):
        i = pl.program_id(0)

        @pl.when(i == 0)
        def _():
            acc_ref[...] = jnp.zeros_like(acc_ref)
            cnt_ref[...] = jnp.zeros_like(cnt_ref)

        seg = lax.broadcasted_iota(jnp.int32, (_NG, nb), 0)
        oh = (seg == b_ref[...].reshape(1, nb)).astype(jnp.float32)
        xcat = jnp.concatenate(
            [x1_ref[...], x2_ref[...], x3_ref[...], x4_ref[...]], axis=1)
        acc_ref[...] += lax.dot_general(
            oh, xcat, (((1,), (0,)), ((), ())),
            preferred_element_type=jnp.float32)
        cnt_ref[...] += jnp.sum(oh, axis=1, keepdims=True)

        @pl.when(i == nblk - 1)
        def _():
            pooled = acc_ref[...] / jnp.maximum(cnt_ref[...], 1.0)
            h1 = jnp.maximum(
                jnp.dot(pooled, w1_ref[...],
                        preferred_element_type=jnp.float32) + b1_ref[...], 0.0)
            h2 = jnp.maximum(
                jnp.dot(h1, w2_ref[...],
                        preferred_element_type=jnp.float32) + b2_ref[...], 0.0)
            h3 = jnp.maximum(
                jnp.dot(h2, w3_ref[...],
                        preferred_element_type=jnp.float32) + b3_ref[...], 0.0)
            z = (jnp.dot(h3, w4_ref[...],
                         preferred_element_type=jnp.float32) + b4_ref[...])
            m = jnp.max(z, axis=1, keepdims=True)
            lse = m + jnp.log(jnp.sum(jnp.exp(z - m), axis=1, keepdims=True))
            o_ref[...] = z - lse

    return pl.pallas_call(
        body,
        grid=(nblk,),
        in_specs=[
            pl.BlockSpec((1, 1, nb), lambda i: (i, 0, 0)),
            pl.BlockSpec((nb, d), lambda i: (i, 0)),
            pl.BlockSpec((nb, d), lambda i: (i, 0)),
            pl.BlockSpec((nb, d), lambda i: (i, 0)),
            pl.BlockSpec((nb, d), lambda i: (i, 0)),
            pl.BlockSpec(w1.shape, lambda i: (0, 0)),
            pl.BlockSpec(b1.shape, lambda i: (0, 0)),
            pl.BlockSpec(w2.shape, lambda i: (0, 0)),
            pl.BlockSpec(b2.shape, lambda i: (0, 0)),
            pl.BlockSpec(w3.shape, lambda i: (0, 0)),
            pl.BlockSpec(b3.shape, lambda i: (0, 0)),
            pl.BlockSpec(w4.shape, lambda i: (0, 0)),
            pl.BlockSpec(b4.shape, lambda i: (0, 0)),
        ],
        out_specs=pl.BlockSpec((_NG, 2), lambda i: (0, 0)),
        out_shape=jax.ShapeDtypeStruct((_NG, 2), jnp.float32),
        scratch_shapes=[
            pltpu.VMEM((_NG, 4 * d), jnp.float32),
            pltpu.VMEM((_NG, 1), jnp.float32),
        ],
    )(batch_row, x1, x2, x3, x4, w1, b1, w2, b2, w3, b3, w4, b4)


def _pad2(w, r, c):
    return jnp.pad(w, ((0, r - w.shape[0]), (0, c - w.shape[1])))


def _padb(b, c):
    return jnp.pad(b, (0, c - b.shape[0])).reshape(1, -1)


def kernel(x, edge_index, edge_attr, batch, params):
    p = params
    c_sz = 80
    eic = jnp.stack([edge_index[0].reshape(-1, c_sz),
                     edge_index[1].reshape(-1, c_sz)], axis=1)

    c1 = p["conv1"]
    ee1 = _edge_mlp(edge_attr,
                    _pad2(c1["be1"]["W"], 3, 16), _padb(c1["be1"]["b"], 16),
                    _pad2(c1["be2"]["W"], 16, 128), _padb(c1["be2"]["b"], 128))
    ees = [
        _edge_mlp(edge_attr, cv["be1"]["W"], cv["be1"]["b"].reshape(1, -1),
                  cv["be2"]["W"], cv["be2"]["b"].reshape(1, -1))
        for cv in (p["conv2"], p["conv3"], p["conv4"])
    ]

    x128 = jnp.pad(x, ((0, 0), (0, 128 - x.shape[1])))

    def layer(xin, cv, ee, bn, pad_in):
        ag = _sc_msgpass(xin, ee, eic)
        if pad_in:
            w1 = _pad2(cv["m1"]["W"], 128, 16)
            b1 = _padb(cv["m1"]["b"], 16)
            w2 = _pad2(cv["m2"]["W"], 16, 128)
        else:
            w1 = cv["m1"]["W"]
            b1 = cv["m1"]["b"].reshape(1, -1)
            w2 = cv["m2"]["W"]
        b2 = cv["m2"]["b"].reshape(1, -1)
        y, st = _node_stats(xin, ag[0], ag[1], w1, b1, w2, b2,
                            cv["eps"].reshape(1, 1))
        return _bn_apply(y, st, bn["g"].reshape(1, -1), bn["b"].reshape(1, -1))

    x1r = layer(x128, p["conv1"], ee1, p["bn1"], True)
    x2r = layer(x1r, p["conv2"], ees[0], p["bn2"], False)
    x3r = layer(x2r, p["conv3"], ees[1], p["bn3"], False)
    x4r = layer(x3r, p["conv4"], ees[2], p["bn4"], False)

    return _pool_head(
        x1r, x2r, x3r, x4r, batch.reshape(-1, 1, 2000),
        p["fc1"]["W"], p["fc1"]["b"].reshape(1, -1),
        p["fc2"]["W"], p["fc2"]["b"].reshape(1, -1),
        p["fc3"]["W"], p["fc3"]["b"].reshape(1, -1),
        p["fc4"]["W"], p["fc4"]["b"].reshape(1, -1))
